# baseline (speedup 1.0000x reference)
import functools

import jax
import jax.numpy as jnp
from jax import lax
from jax.experimental import pallas as pl
from jax.experimental.pallas import tpu as pltpu
from jax.experimental.pallas import tpu_sc as plsc

NUM_CORES = 2
NUM_SUBCORES = 16
LANES = 16
NW = NUM_CORES * NUM_SUBCORES
R = 5

NEG_INF = float("-inf")
POS_INF = float("inf")

B_SC = 128
RB_TC = 128


def _make_sc_kernel(B, L, n_rows):
    rows_per = n_rows // NW
    mesh = plsc.VectorSubcoreMesh(
        core_axis_name="c", subcore_axis_name="s",
        num_cores=NUM_CORES, num_subcores=NUM_SUBCORES)

    @functools.partial(
        pl.kernel,
        out_type=jax.ShapeDtypeStruct((n_rows * 2 * R,), jnp.float32),
        mesh=mesh,
        compiler_params=pltpu.CompilerParams(needs_layout_passes=False),
        scratch_types=[
            pltpu.VMEM((rows_per, L), jnp.float32),
            pltpu.VMEM((rows_per * 2 * R + LANES,), jnp.float32),
            pltpu.VMEM((B,), jnp.int32),
        ],
    )
    def k(x_hbm, len_hbm, out_hbm, xbuf, obuf, lenbuf):
        wid = lax.axis_index("s") * NUM_CORES + lax.axis_index("c")
        base = wid * rows_per
        pltpu.sync_copy(len_hbm, lenbuf)
        pltpu.sync_copy(x_hbm.at[pl.ds(base, rows_per)], xbuf)

        iota = lax.iota(jnp.int32, LANES)
        minf = jnp.full((LANES,), NEG_INF, jnp.float32)
        pinf = jnp.full((LANES,), POS_INF, jnp.float32)

        def do_row(r):
            g = base + r
            lv = lenbuf[pl.ds((g // LANES) * LANES, LANES)]
            lf = jnp.where(iota == g % LANES, lv.astype(jnp.float32), 0.0)
            l = jnp.max(lf, axis=0).astype(jnp.int32)
            out_vec = pinf
            for i in range(R):
                s = (i * l) // R
                e = ((i + 1) * l + (R - 1)) // R
                vfirst = s // LANES
                vlast = (e - 1) // LANES
                xf = xbuf[r, pl.ds(vfirst * LANES, LANES)]
                idxf = vfirst * LANES + iota
                mf = (idxf >= s) & (idxf < e)
                xl = xbuf[r, pl.ds(vlast * LANES, LANES)]
                idxl = vlast * LANES + iota
                ml = (idxl >= s) & (idxl < e)
                acc = (jnp.where(mf, xf, minf), jnp.where(ml, xl, minf),
                       minf, minf,
                       jnp.where(mf, xf, pinf), jnp.where(ml, xl, pinf),
                       pinf, pinf)
                lo = vfirst + 1
                hi = vlast
                num = jnp.maximum(hi - lo, 0)
                trips = (num + 3) // 4

                @plsc.parallel_loop(0, trips, unroll=2, carry=acc)
                def mloop(t, c):
                    a0, a1, a2, a3, b0, b1, b2, b3 = c
                    v0 = lo + t * 4
                    v1 = jnp.minimum(v0 + 1, hi - 1)
                    v2 = jnp.minimum(v0 + 2, hi - 1)
                    v3 = jnp.minimum(v0 + 3, hi - 1)
                    x0 = xbuf[r, pl.ds(v0 * LANES, LANES)]
                    x1 = xbuf[r, pl.ds(v1 * LANES, LANES)]
                    x2 = xbuf[r, pl.ds(v2 * LANES, LANES)]
                    x3 = xbuf[r, pl.ds(v3 * LANES, LANES)]
                    return (jnp.maximum(a0, x0), jnp.maximum(a1, x1),
                            jnp.maximum(a2, x2), jnp.maximum(a3, x3),
                            jnp.minimum(b0, x0), jnp.minimum(b1, x1),
                            jnp.minimum(b2, x2), jnp.minimum(b3, x3))

                a0, a1, a2, a3, b0, b1, b2, b3 = mloop
                amax = jnp.maximum(jnp.maximum(a0, a1), jnp.maximum(a2, a3))
                amin = jnp.minimum(jnp.minimum(b0, b1), jnp.minimum(b2, b3))
                mx = jnp.max(amax, axis=0)
                mn = jnp.min(amin, axis=0)
                out_vec = jnp.where(iota == i, mx, out_vec)
                out_vec = jnp.where(iota == R + i, mn, out_vec)
            obuf[pl.ds(r * 2 * R, LANES)] = lax.sort(out_vec)

        def row_body(r, _):
            do_row(r)
            return 0

        lax.fori_loop(0, rows_per, row_body, 0)
        pltpu.sync_copy(obuf.at[pl.ds(0, rows_per * 2 * R)],
                        out_hbm.at[pl.ds(base * 2 * R, rows_per * 2 * R)])

    return k


def _bitonic_pairs(n):
    ops = []
    k = 2
    while k <= n:
        j = k // 2
        while j >= 1:
            for i in range(n):
                m = i ^ j
                if m > i:
                    ops.append((i, m, (i & k) == 0))
            j //= 2
        k *= 2
    return ops


def _tc_body(x_ref, len_ref, out_ref):
    x = x_ref[...]
    l = len_ref[...].astype(jnp.int32)
    rb, L = x.shape
    j = lax.broadcasted_iota(jnp.int32, (rb, L), 1)
    cols = []
    mins = []
    for i in range(R):
        s = (i * l) // R
        e = ((i + 1) * l + (R - 1)) // R
        m = (j >= s) & (j < e)
        cols.append(jnp.max(jnp.where(m, x, NEG_INF), axis=1, keepdims=True))
        mins.append(jnp.min(jnp.where(m, x, POS_INF), axis=1, keepdims=True))
    cols.extend(mins)
    cols.extend([jnp.full((rb, 1), POS_INF, jnp.float32)] * 6)
    for a, b, asc in _bitonic_pairs(16):
        lo = jnp.minimum(cols[a], cols[b])
        hi = jnp.maximum(cols[a], cols[b])
        cols[a], cols[b] = (lo, hi) if asc else (hi, lo)
    out_ref[...] = jnp.concatenate(cols[: 2 * R], axis=1)


def _make_tc_kernel(n_rows, L, row0):
    grid = (n_rows // RB_TC,)
    g0 = row0 // RB_TC
    return pl.pallas_call(
        _tc_body,
        grid=grid,
        in_specs=[
            pl.BlockSpec((RB_TC, L), lambda g: (g + g0, 0)),
            pl.BlockSpec((RB_TC, 1), lambda g: (g + g0, 0)),
        ],
        out_specs=pl.BlockSpec((RB_TC, 2 * R), lambda g: (g, 0)),
        out_shape=jax.ShapeDtypeStruct((n_rows, 2 * R), jnp.float32),
    )


@jax.jit
def kernel(inputs, lengths):
    B, L = inputs.shape
    lengths = lengths.astype(jnp.int32)
    sc_flat = _make_sc_kernel(B, L, B_SC)(inputs, lengths)
    tc_out = _make_tc_kernel(B - B_SC, L, B_SC)(
        inputs, lengths[:, None].astype(jnp.float32))
    return jnp.concatenate([sc_flat.reshape(B_SC, 2 * R), tc_out], axis=0)
